# baseline (speedup 1.0000x reference)
import functools

import jax
import jax.numpy as jnp
from jax import lax
from jax.experimental import pallas as pl
from jax.experimental.pallas import tpu as pltpu
from jax.experimental.pallas import tpu_sc as plsc

D_MODEL = 64
SEQ = 200
BATCH = 4096
BB = 128
NUM_CORES = 2
NUM_SUBCORES = 16
NUM_WORKERS = NUM_CORES * NUM_SUBCORES


def _sc_embed(x_lin, w, pos_lin):
    mesh = plsc.VectorSubcoreMesh(
        core_axis_name="c", subcore_axis_name="s",
        num_cores=NUM_CORES, num_subcores=NUM_SUBCORES)

    @functools.partial(
        pl.kernel,
        out_type=jax.ShapeDtypeStruct(
            (SEQ, D_MODEL // 8, BATCH // BB, 8, BB), jnp.float32),
        mesh=mesh,
        compiler_params=pltpu.CompilerParams(
            use_tc_tiling_on_sc=False, needs_layout_passes=False),
        scratch_types=[
            pltpu.VMEM((2, BB), jnp.int32),
            pltpu.VMEM((2, BB, D_MODEL), jnp.float32),
            pltpu.VMEM((2, D_MODEL, BB), jnp.float32),
            pltpu.VMEM((SEQ * D_MODEL,), jnp.float32),
            pltpu.SemaphoreType.DMA((2,)),
            pltpu.SemaphoreType.DMA((2,)),
        ],
    )
    def body(x_hbm, w_hbm, pos_hbm, out_hbm, idx_v, rows_v, trans_v, pos_v,
             gsem, osem):
        wid = lax.axis_index("s") * NUM_CORES + lax.axis_index("c")
        pltpu.sync_copy(pos_hbm, pos_v)

        def start_gather(p, t):
            pltpu.sync_copy(x_hbm.at[pl.ds(t * BATCH + wid * BB, BB)],
                            idx_v.at[p])
            pltpu.async_copy(w_hbm.at[idx_v.at[p]], rows_v.at[p], gsem.at[p])

        def wait_gather(p):
            pltpu.make_async_copy(
                w_hbm.at[idx_v.at[p]], rows_v.at[p], gsem.at[p]).wait()

        def start_out(p, t):
            for g in range(D_MODEL // 8):
                pltpu.async_copy(trans_v.at[p, pl.ds(8 * g, 8)],
                                 out_hbm.at[t, g, wid], osem.at[p])

        def wait_out(p, t):
            for g in range(D_MODEL // 8):
                pltpu.make_async_copy(trans_v.at[p, pl.ds(8 * g, 8)],
                                      out_hbm.at[t, g, wid], osem.at[p]).wait()

        def transpose_add(p, t):
            row_base = [lax.iota(jnp.int32, 16) + 16 * j for j in range(8)]

            @plsc.parallel_loop(0, D_MODEL, step=1)
            def _(c):
                col = jnp.full((16,), c, jnp.int32)
                pv = plsc.load_gather(pos_v, [col + t * D_MODEL])
                for j in range(BB // 16):
                    v = plsc.load_gather(rows_v.at[p], [row_base[j], col])
                    trans_v[p, c, pl.ds(16 * j, 16)] = v + pv

        start_gather(0, 0)

        def step(t2, carry):
            for p in range(2):
                t = 2 * t2 + p
                wait_gather(p)

                @pl.when(t < SEQ - 1)
                def _():
                    start_gather(1 - p, t + 1)

                @pl.when(t >= 2)
                def _():
                    wait_out(p, t - 2)

                transpose_add(p, t)
                start_out(p, t)
            return carry

        lax.fori_loop(0, SEQ // 2, step, 0)
        wait_out(0, SEQ - 2)
        wait_out(1, SEQ - 1)

    return body(x_lin, w, pos_lin)


def kernel(X, W, pos):
    batch, seq = X.shape
    x_lin = X.T.reshape(-1).astype(jnp.int32)
    out5 = _sc_embed(x_lin, W, pos[:seq].reshape(-1))
    return out5.transpose(2, 4, 0, 1, 3).reshape(batch, seq, D_MODEL)


# speedup vs baseline: 2.3919x; 2.3919x over previous
import functools

import jax
import jax.numpy as jnp
from jax import lax
from jax.experimental import pallas as pl
from jax.experimental.pallas import tpu as pltpu
from jax.experimental.pallas import tpu_sc as plsc

D_MODEL = 64
SEQ = 200
BATCH = 4096
BB = 128
NUM_CORES = 2
NUM_SUBCORES = 16
NUM_WORKERS = NUM_CORES * NUM_SUBCORES

VOCAB_ROWS = 1000000
WS = 65
VB = 128
FULL_VB = VOCAB_ROWS // VB
VTAIL = VOCAB_ROWS - FULL_VB * VB


def _mesh():
    return plsc.VectorSubcoreMesh(
        core_axis_name="c", subcore_axis_name="s",
        num_cores=NUM_CORES, num_subcores=NUM_SUBCORES)


def _sc_relayout(wt, tail_lin):
    @functools.partial(
        pl.kernel,
        out_type=jax.ShapeDtypeStruct((VOCAB_ROWS * D_MODEL,), jnp.float32),
        mesh=_mesh(),
        compiler_params=pltpu.CompilerParams(
            use_tc_tiling_on_sc=True, needs_layout_passes=False),
        scratch_types=[
            pltpu.VMEM((D_MODEL, VB), jnp.float32),
            pltpu.VMEM((D_MODEL, VB), jnp.float32),
            pltpu.VMEM((VB * WS,), jnp.float32),
            pltpu.VMEM((VB * WS,), jnp.float32),
            pltpu.VMEM((VB * D_MODEL,), jnp.float32),
            pltpu.VMEM((VB * D_MODEL,), jnp.float32),
            pltpu.SemaphoreType.DMA((2,)),
            pltpu.SemaphoreType.DMA((2,)),
        ],
    )
    def body(wt_hbm, tail_hbm, out_hbm, vin0, vin1, vout0, vout1,
             vcomp0, vcomp1, isem, osem):
        vin = [vin0, vin1]
        vout = [vout0, vout1]
        vcomp = [vcomp0, vcomp1]
        wid = lax.axis_index("s") * NUM_CORES + lax.axis_index("c")
        nblk = (FULL_VB - wid + NUM_WORKERS - 1) // NUM_WORKERS

        def v0(i):
            return (wid + i * NUM_WORKERS) * VB

        def start_in(p, v):
            pltpu.async_copy(wt_hbm.at[pl.ds(0, D_MODEL), pl.ds(v, VB)],
                             vin[p], isem.at[p])

        def wait_in(p, v):
            pltpu.make_async_copy(
                wt_hbm.at[pl.ds(0, D_MODEL), pl.ds(v, VB)],
                vin[p], isem.at[p]).wait()

        def start_out(p, v):
            pltpu.async_copy(vcomp[p],
                             out_hbm.at[pl.ds(v * D_MODEL, VB * D_MODEL)],
                             osem.at[p])

        def wait_out(p, v):
            pltpu.make_async_copy(
                vcomp[p], out_hbm.at[pl.ds(v * D_MODEL, VB * D_MODEL)],
                osem.at[p]).wait()

        l_base = [(lax.iota(jnp.int32, 16) + 16 * m) * WS
                  for m in range(VB // 16)]

        def transpose(p):
            @plsc.parallel_loop(0, D_MODEL, step=1, unroll=4)
            def _(c):
                csp = jnp.full((16,), c, jnp.int32)
                for m in range(VB // 16):
                    vals = vin[p][c, pl.ds(16 * m, 16)]
                    plsc.store_scatter(vout[p], [l_base[m] + csp], vals)

            @plsc.parallel_loop(0, VB, step=1, unroll=4)
            def _(r):
                for k in range(D_MODEL // 16):
                    vcomp[p][pl.ds(r * D_MODEL + 16 * k, 16)] = (
                        vout[p][pl.ds(r * WS + 16 * k, 16)])

        for p in range(2):
            @pl.when(p < nblk)
            def _():
                start_in(p, v0(p))

        def step(i, carry):
            for p in range(2):
                n = 2 * i + p

                @pl.when(n < nblk)
                def _():
                    v = v0(n)
                    wait_in(p, v)

                    @pl.when(n >= 2)
                    def _():
                        wait_out(p, v0(n - 2))

                    transpose(p)
                    start_out(p, v)

                    @pl.when(n + 2 < nblk)
                    def _():
                        start_in(p, v0(n + 2))
            return carry

        lax.fori_loop(0, (FULL_VB // NUM_WORKERS + 2) // 2, step, 0)

        for p in range(2):
            m = nblk - 1
            n_p = m - lax.rem(m - p + 2, 2)

            @pl.when(n_p >= 0)
            def _():
                wait_out(p, v0(n_p))

        @pl.when(wid == 0)
        def _():
            v = FULL_VB * VB
            pltpu.sync_copy(tail_hbm, vout0.at[pl.ds(0, VTAIL * D_MODEL)])
            pltpu.sync_copy(vout0.at[pl.ds(0, VTAIL * D_MODEL)],
                            out_hbm.at[pl.ds(v * D_MODEL, VTAIL * D_MODEL)])

    return body(wt, tail_lin)


def _sc_embed(x_lin, w, pos_lin):
    @functools.partial(
        pl.kernel,
        out_type=jax.ShapeDtypeStruct(
            (SEQ, D_MODEL // 8, BATCH // BB, 8, BB), jnp.float32),
        mesh=_mesh(),
        compiler_params=pltpu.CompilerParams(
            use_tc_tiling_on_sc=False, needs_layout_passes=False),
        scratch_types=[
            pltpu.VMEM((2, BB), jnp.int32),
            pltpu.VMEM((2, BB, D_MODEL), jnp.float32),
            pltpu.VMEM((2, D_MODEL, BB + 1), jnp.float32),
            pltpu.VMEM((SEQ * D_MODEL,), jnp.float32),
            pltpu.SemaphoreType.DMA((2,)),
            pltpu.SemaphoreType.DMA((2,)),
        ],
    )
    def body(x_hbm, w_hbm, pos_hbm, out_hbm, idx_v, rows_v, trans_v, pos_v,
             gsem, osem):
        wid = lax.axis_index("s") * NUM_CORES + lax.axis_index("c")
        pltpu.sync_copy(pos_hbm, pos_v)

        def start_gather(p, t):
            pltpu.sync_copy(x_hbm.at[pl.ds(t * BATCH + wid * BB, BB)],
                            idx_v.at[p])
            pltpu.async_copy(w_hbm.at[idx_v.at[p]], rows_v.at[p], gsem.at[p])

        def wait_gather(p):
            pltpu.make_async_copy(
                w_hbm.at[idx_v.at[p]], rows_v.at[p], gsem.at[p]).wait()

        def start_out(p, t):
            for g in range(D_MODEL // 8):
                pltpu.async_copy(trans_v.at[p, pl.ds(8 * g, 8), pl.ds(0, BB)],
                                 out_hbm.at[t, g, wid], osem.at[p])

        def wait_out(p, t):
            for g in range(D_MODEL // 8):
                pltpu.make_async_copy(trans_v.at[p, pl.ds(8 * g, 8), pl.ds(0, BB)],
                                      out_hbm.at[t, g, wid], osem.at[p]).wait()

        c_base = [lax.iota(jnp.int32, 16) + 16 * k for k in range(D_MODEL // 16)]

        def transpose_add(p, t):
            posk = [pos_v[pl.ds(t * D_MODEL + 16 * k, 16)]
                    for k in range(D_MODEL // 16)]

            @plsc.parallel_loop(0, BB, step=1, unroll=4)
            def _(l):
                lsp = jnp.full((16,), l, jnp.int32)
                for k in range(D_MODEL // 16):
                    vals = rows_v[p, l, pl.ds(16 * k, 16)] + posk[k]
                    plsc.store_scatter(trans_v.at[p], [c_base[k], lsp], vals)

        start_gather(0, 0)

        def step(t2, carry):
            for p in range(2):
                t = 2 * t2 + p
                wait_gather(p)

                @pl.when(t < SEQ - 1)
                def _():
                    start_gather(1 - p, t + 1)

                @pl.when(t >= 2)
                def _():
                    wait_out(p, t - 2)

                transpose_add(p, t)
                start_out(p, t)
            return carry

        lax.fori_loop(0, SEQ // 2, step, 0)
        wait_out(0, SEQ - 2)
        wait_out(1, SEQ - 1)

    return body(x_lin, w, pos_lin)


def kernel(X, W, pos):
    batch, seq = X.shape
    x_lin = X.T.reshape(-1).astype(jnp.int32)
    tail_lin = W[FULL_VB * VB:].reshape(-1)
    w_lin = _sc_relayout(W.T, tail_lin).reshape(VOCAB_ROWS, D_MODEL)
    out5 = _sc_embed(x_lin, w_lin, pos[:seq].reshape(-1))
    return out5.transpose(2, 4, 0, 1, 3).reshape(batch, seq, D_MODEL)
